# baseline (speedup 1.0000x reference)
import jax
import jax.numpy as jnp
from jax import lax
from jax.experimental import pallas as pl
from jax.experimental.pallas import tpu as pltpu
from jax.experimental.pallas import tpu_sc as plsc

N = 10000
NPAD = 10240
E = 320000
D = 128
NHID = 64
H1 = 8
NLAB = 16

CH = 2000
NCH1 = E // CH
NCH2 = (E // 2) // CH

_SC_MESH = plsc.VectorSubcoreMesh(core_axis_name="c", subcore_axis_name="s")
_SC_PARAMS = pltpu.CompilerParams(needs_layout_passes=False,
                                  use_tc_tiling_on_sc=False)
_F32 = jnp.float32


def _leaky_exp(es, ed):
    e = es + ed
    e = jnp.maximum(e, 0.2 * e)
    return jnp.exp(e)


def _tc1_body(w_ref, bd_ref, x_ref, h_ref, esd_ref):
    hb = jnp.dot(w_ref[...], x_ref[...], preferred_element_type=_F32)
    h_ref[...] = hb
    esd_ref[...] = jnp.dot(bd_ref[...], hb, preferred_element_type=_F32)


def _tc1(w1flat, bd, xT):
    return pl.pallas_call(
        _tc1_body,
        grid=(NPAD // 1024,),
        in_specs=[
            pl.BlockSpec((H1 * NHID, D), lambda i: (0, 0)),
            pl.BlockSpec((2 * H1, H1 * NHID), lambda i: (0, 0)),
            pl.BlockSpec((D, 1024), lambda i: (0, i)),
        ],
        out_specs=[
            pl.BlockSpec((H1 * NHID, 1024), lambda i: (0, i)),
            pl.BlockSpec((2 * H1, 1024), lambda i: (0, i)),
        ],
        out_shape=[
            jax.ShapeDtypeStruct((H1 * NHID, NPAD), _F32),
            jax.ShapeDtypeStruct((2 * H1, NPAD), _F32),
        ],
    )(w1flat, bd, xT)


def _tc2_body(w_ref, a_ref, h_ref, h2_ref, esd_ref):
    h2 = jnp.dot(w_ref[...], h_ref[...], preferred_element_type=_F32)
    h2_ref[...] = h2
    esd_ref[...] = jnp.dot(a_ref[...], h2, preferred_element_type=_F32)


def _tc2(w2T, a2, h1T):
    return pl.pallas_call(
        _tc2_body,
        grid=(NPAD // 1024,),
        in_specs=[
            pl.BlockSpec((NLAB, H1 * NHID), lambda i: (0, 0)),
            pl.BlockSpec((2, NLAB), lambda i: (0, 0)),
            pl.BlockSpec((H1 * NHID, 1024), lambda i: (0, i)),
        ],
        out_specs=[
            pl.BlockSpec((NLAB, 1024), lambda i: (0, i)),
            pl.BlockSpec((2, 1024), lambda i: (0, i)),
        ],
        out_shape=[
            jax.ShapeDtypeStruct((NLAB, NPAD), _F32),
            jax.ShapeDtypeStruct((2, NPAD), _F32),
        ],
    )(w2T, a2, h1T)


def _tc3_body(acc_ref, ss_ref, o_ref):
    a = acc_ref[...]
    s = ss_ref[0:1, :] + ss_ref[1:2, :] + 1e-9
    h2 = (a[0:NLAB] + a[NLAB:2 * NLAB]) / s
    m = jnp.max(h2, axis=0, keepdims=True)
    ex = jnp.exp(h2 - m)
    p = ex / jnp.sum(ex, axis=0, keepdims=True)
    o_ref[...] = p.T


def _tc3(acc2, ssum2):
    return pl.pallas_call(
        _tc3_body,
        grid=(NPAD // 1024,),
        in_specs=[
            pl.BlockSpec((2 * NLAB, 1024), lambda i: (0, i)),
            pl.BlockSpec((2, 1024), lambda i: (0, i)),
        ],
        out_specs=pl.BlockSpec((1024, NLAB), lambda i: (i, 0)),
        out_shape=jax.ShapeDtypeStruct((NPAD, NLAB), _F32),
    )(acc2, ssum2)


def _sc1(hT, esd, src, dst):
    @pl.kernel(
        out_type=jax.ShapeDtypeStruct((H1 * NHID, NPAD), _F32),
        mesh=_SC_MESH,
        scratch_types=[
            pltpu.VMEM((N,), _F32),
            pltpu.VMEM((N,), _F32),
            pltpu.VMEM((4 * N,), _F32),
            pltpu.VMEM((4 * N,), _F32),
            pltpu.VMEM((N,), _F32),
            pltpu.VMEM((2, CH), jnp.int32),
            pltpu.VMEM((2, CH), jnp.int32),
            pltpu.SemaphoreType.DMA,
            pltpu.SemaphoreType.DMA,
        ],
        compiler_params=_SC_PARAMS,
    )
    def k(h_hbm, esd_hbm, src_hbm, dst_hbm, out_hbm,
          es_v, ed_v, h_v, acc_v, ssum_v, sbuf, dbuf, sem0, sem1):
        wid = lax.axis_index("s") * 2 + lax.axis_index("c")
        sems = (sem0, sem1)

        def fire(chunk, b, sem):
            off = chunk * CH
            pltpu.async_copy(src_hbm.at[pl.ds(off, CH)], sbuf.at[b], sem)
            pltpu.async_copy(dst_hbm.at[pl.ds(off, CH)], dbuf.at[b], sem)

        def drain(chunk, b, sem):
            off = chunk * CH
            pltpu.make_async_copy(src_hbm.at[pl.ds(off, CH)], sbuf.at[b], sem).wait()
            pltpu.make_async_copy(dst_hbm.at[pl.ds(off, CH)], dbuf.at[b], sem).wait()

        def process(b):
            sb = sbuf.at[b]
            db = dbuf.at[b]

            @plsc.parallel_loop(0, CH, step=16, unroll=6)
            def _(j):
                s = sb[pl.ds(j, 16)]
                t = db[pl.ds(j, 16)]
                ex = _leaky_exp(plsc.load_gather(es_v, [s]),
                                plsc.load_gather(ed_v, [t]))
                plsc.addupdate_scatter(ssum_v, [t], ex)
                for d in range(4):
                    hg = plsc.load_gather(h_v, [s + d * N])
                    plsc.addupdate_scatter(acc_v, [t + d * N], ex * hg)

        for p in range(4):
            head = 2 * p + wid // 16
            rb = p * 128 + wid * 4
            pltpu.sync_copy(esd_hbm.at[head, pl.ds(0, N)], es_v)
            pltpu.sync_copy(esd_hbm.at[H1 + head, pl.ds(0, N)], ed_v)
            for d in range(4):
                pltpu.sync_copy(h_hbm.at[rb + d, pl.ds(0, N)],
                                h_v.at[pl.ds(d * N, N)])
            fire(0, 0, sems[0])

            @plsc.parallel_loop(0, N, step=16, unroll=4)
            def _(j):
                ex = _leaky_exp(es_v[pl.ds(j, 16)], ed_v[pl.ds(j, 16)])
                ssum_v[pl.ds(j, 16)] = ex
                for d in range(4):
                    acc_v[pl.ds(j + d * N, 16)] = ex * h_v[pl.ds(j + d * N, 16)]

            @pl.loop(0, NCH1, step=2)
            def _(g):
                fire(g + 1, 1, sems[1])
                drain(g, 0, sems[0])
                process(0)

                @pl.when(g + 2 < NCH1)
                def _():
                    fire(g + 2, 0, sems[0])

                drain(g + 1, 1, sems[1])
                process(1)

            @plsc.parallel_loop(0, N, step=16, unroll=4)
            def _(j):
                inv = 1.0 / (ssum_v[pl.ds(j, 16)] + 1e-9)
                for d in range(4):
                    v = acc_v[pl.ds(j + d * N, 16)] * inv
                    acc_v[pl.ds(j + d * N, 16)] = jnp.where(
                        v > 0, v, jnp.exp(v) - 1.0)

            for d in range(4):
                pltpu.sync_copy(acc_v.at[pl.ds(d * N, N)],
                                out_hbm.at[rb + d, pl.ds(0, N)])

    return k(hT, esd, src, dst)


def _sc2(h2T, esd2, src, dst):
    @pl.kernel(
        out_type=[
            jax.ShapeDtypeStruct((2 * NLAB, NPAD), _F32),
            jax.ShapeDtypeStruct((2, NPAD), _F32),
        ],
        mesh=_SC_MESH,
        scratch_types=[
            pltpu.VMEM((N,), _F32),
            pltpu.VMEM((N,), _F32),
            pltpu.VMEM((N,), _F32),
            pltpu.VMEM((N,), _F32),
            pltpu.VMEM((N,), _F32),
            pltpu.VMEM((2, CH), jnp.int32),
            pltpu.VMEM((2, CH), jnp.int32),
            pltpu.SemaphoreType.DMA,
            pltpu.SemaphoreType.DMA,
        ],
        compiler_params=_SC_PARAMS,
    )
    def k(h_hbm, esd_hbm, src_hbm, dst_hbm, acc_hbm, ss_hbm,
          es_v, ed_v, h_v, acc_v, ssum_v, sbuf, dbuf, sem0, sem1):
        wid = lax.axis_index("s") * 2 + lax.axis_index("c")
        d = wid % NLAB
        half = wid // NLAB
        ebase = half * NCH2
        sems = (sem0, sem1)

        def fire(chunk, b, sem):
            off = (ebase + chunk) * CH
            pltpu.async_copy(src_hbm.at[pl.ds(off, CH)], sbuf.at[b], sem)
            pltpu.async_copy(dst_hbm.at[pl.ds(off, CH)], dbuf.at[b], sem)

        def drain(chunk, b, sem):
            off = (ebase + chunk) * CH
            pltpu.make_async_copy(src_hbm.at[pl.ds(off, CH)], sbuf.at[b], sem).wait()
            pltpu.make_async_copy(dst_hbm.at[pl.ds(off, CH)], dbuf.at[b], sem).wait()

        def process(b):
            sb = sbuf.at[b]
            db = dbuf.at[b]

            @plsc.parallel_loop(0, CH, step=16, unroll=6)
            def _(j):
                s = sb[pl.ds(j, 16)]
                t = db[pl.ds(j, 16)]
                ex = _leaky_exp(plsc.load_gather(es_v, [s]),
                                plsc.load_gather(ed_v, [t]))
                plsc.addupdate_scatter(ssum_v, [t], ex)
                hg = plsc.load_gather(h_v, [s])
                plsc.addupdate_scatter(acc_v, [t], ex * hg)

        pltpu.sync_copy(esd_hbm.at[0, pl.ds(0, N)], es_v)
        pltpu.sync_copy(esd_hbm.at[1, pl.ds(0, N)], ed_v)
        pltpu.sync_copy(h_hbm.at[d, pl.ds(0, N)], h_v)
        fire(0, 0, sems[0])

        flag = jnp.where(half == 0, 1.0, 0.0).astype(_F32)

        @plsc.parallel_loop(0, N, step=16, unroll=4)
        def _(j):
            ex = flag * _leaky_exp(es_v[pl.ds(j, 16)], ed_v[pl.ds(j, 16)])
            ssum_v[pl.ds(j, 16)] = ex
            acc_v[pl.ds(j, 16)] = ex * h_v[pl.ds(j, 16)]

        @pl.loop(0, NCH2, step=2)
        def _(g):
            fire(g + 1, 1, sems[1])
            drain(g, 0, sems[0])
            process(0)

            @pl.when(g + 2 < NCH2)
            def _():
                fire(g + 2, 0, sems[0])

            drain(g + 1, 1, sems[1])
            process(1)

        pltpu.sync_copy(acc_v, acc_hbm.at[wid, pl.ds(0, N)])

        @pl.when(d == 0)
        def _():
            pltpu.sync_copy(ssum_v, ss_hbm.at[half, pl.ds(0, N)])

    return k(h2T, esd2, src, dst)


def kernel(x, edge_index, W1, a1s, a1d, W2, a2s, a2d):
    xT = jnp.pad(x.T, ((0, 0), (0, NPAD - N)))
    src = edge_index[0]
    dst = edge_index[1]

    w1flat = jnp.transpose(W1, (0, 2, 1)).reshape(H1 * NHID, D)
    eye = jnp.eye(H1, dtype=_F32)
    bd_s = jnp.einsum("ij,jf->ijf", eye, a1s).reshape(H1, H1 * NHID)
    bd_d = jnp.einsum("ij,jf->ijf", eye, a1d).reshape(H1, H1 * NHID)
    bd = jnp.concatenate([bd_s, bd_d], axis=0)

    hT, esd = _tc1(w1flat, bd, xT)
    h1T = _sc1(hT, esd, src, dst)

    w2T = W2[0].T
    a2 = jnp.concatenate([a2s, a2d], axis=0)
    h2T, esd2 = _tc2(w2T, a2, h1T)
    acc2, ssum2 = _sc2(h2T, esd2, src, dst)

    out = _tc3(acc2, ssum2)
    return out[:N]


# speedup vs baseline: 1.2785x; 1.2785x over previous
import jax
import jax.numpy as jnp
from jax import lax
from jax.experimental import pallas as pl
from jax.experimental.pallas import tpu as pltpu
from jax.experimental.pallas import tpu_sc as plsc

N = 10000
NPAD = 10240
E = 320000
D = 128
NHID = 64
H1 = 8
NLAB = 16

CH = 4000
NCH1 = E // CH
NCH2 = (E // 2) // CH

_SC_MESH = plsc.VectorSubcoreMesh(core_axis_name="c", subcore_axis_name="s")
_SC_PARAMS = pltpu.CompilerParams(needs_layout_passes=False,
                                  use_tc_tiling_on_sc=False)
_F32 = jnp.float32


def _leaky_exp(es, ed):
    e = es + ed
    e = jnp.maximum(e, 0.2 * e)
    return jnp.exp(e)


def _tc1_body(w_ref, bd_ref, x_ref, h_ref, esd_ref):
    hb = jnp.dot(w_ref[...], x_ref[...], preferred_element_type=_F32)
    h_ref[...] = hb
    esd_ref[...] = jnp.dot(bd_ref[...], hb, preferred_element_type=_F32)


def _tc1(w1flat, bd, xT):
    return pl.pallas_call(
        _tc1_body,
        grid=(NPAD // 1024,),
        in_specs=[
            pl.BlockSpec((H1 * NHID, D), lambda i: (0, 0)),
            pl.BlockSpec((2 * H1, H1 * NHID), lambda i: (0, 0)),
            pl.BlockSpec((D, 1024), lambda i: (0, i)),
        ],
        out_specs=[
            pl.BlockSpec((H1 * NHID, 1024), lambda i: (0, i)),
            pl.BlockSpec((2 * H1, 1024), lambda i: (0, i)),
        ],
        out_shape=[
            jax.ShapeDtypeStruct((H1 * NHID, NPAD), _F32),
            jax.ShapeDtypeStruct((2 * H1, NPAD), _F32),
        ],
    )(w1flat, bd, xT)


def _tc2_body(w_ref, a_ref, h_ref, h2_ref, esd_ref):
    h2 = jnp.dot(w_ref[...], h_ref[...], preferred_element_type=_F32)
    h2_ref[...] = h2
    esd_ref[...] = jnp.dot(a_ref[...], h2, preferred_element_type=_F32)


def _tc2(w2T, a2, h1T):
    return pl.pallas_call(
        _tc2_body,
        grid=(NPAD // 1024,),
        in_specs=[
            pl.BlockSpec((NLAB, H1 * NHID), lambda i: (0, 0)),
            pl.BlockSpec((2, NLAB), lambda i: (0, 0)),
            pl.BlockSpec((H1 * NHID, 1024), lambda i: (0, i)),
        ],
        out_specs=[
            pl.BlockSpec((NLAB, 1024), lambda i: (0, i)),
            pl.BlockSpec((2, 1024), lambda i: (0, i)),
        ],
        out_shape=[
            jax.ShapeDtypeStruct((NLAB, NPAD), _F32),
            jax.ShapeDtypeStruct((2, NPAD), _F32),
        ],
    )(w2T, a2, h1T)


def _tc3_body(acc_ref, ss_ref, o_ref):
    a = acc_ref[...]
    s = ss_ref[0:1, :] + ss_ref[1:2, :] + 1e-9
    h2 = (a[0:NLAB] + a[NLAB:2 * NLAB]) / s
    m = jnp.max(h2, axis=0, keepdims=True)
    ex = jnp.exp(h2 - m)
    p = ex / jnp.sum(ex, axis=0, keepdims=True)
    o_ref[...] = p.T


def _tc3(acc2, ssum2):
    return pl.pallas_call(
        _tc3_body,
        grid=(NPAD // 1024,),
        in_specs=[
            pl.BlockSpec((2 * NLAB, 1024), lambda i: (0, i)),
            pl.BlockSpec((2, 1024), lambda i: (0, i)),
        ],
        out_specs=pl.BlockSpec((1024, NLAB), lambda i: (i, 0)),
        out_shape=jax.ShapeDtypeStruct((NPAD, NLAB), _F32),
    )(acc2, ssum2)


def _sc1(hT, esd, src, dst):
    @pl.kernel(
        out_type=jax.ShapeDtypeStruct((H1 * NHID, NPAD), _F32),
        mesh=_SC_MESH,
        scratch_types=[
            pltpu.VMEM((N,), _F32),
            pltpu.VMEM((N,), _F32),
            pltpu.VMEM((4 * N,), _F32),
            pltpu.VMEM((4 * N,), _F32),
            pltpu.VMEM((N,), _F32),
            pltpu.VMEM((2, CH), jnp.int32),
            pltpu.VMEM((2, CH), jnp.int32),
            pltpu.SemaphoreType.DMA,
            pltpu.SemaphoreType.DMA,
        ],
        compiler_params=_SC_PARAMS,
    )
    def k(h_hbm, esd_hbm, src_hbm, dst_hbm, out_hbm,
          es_v, ed_v, h_v, acc_v, ssum_v, sbuf, dbuf, sem0, sem1):
        wid = lax.axis_index("s") * 2 + lax.axis_index("c")
        sems = (sem0, sem1)

        def fire(chunk, b, sem):
            off = chunk * CH
            pltpu.async_copy(src_hbm.at[pl.ds(off, CH)], sbuf.at[b], sem)
            pltpu.async_copy(dst_hbm.at[pl.ds(off, CH)], dbuf.at[b], sem)

        def drain(chunk, b, sem):
            off = chunk * CH
            pltpu.make_async_copy(src_hbm.at[pl.ds(off, CH)], sbuf.at[b], sem).wait()
            pltpu.make_async_copy(dst_hbm.at[pl.ds(off, CH)], dbuf.at[b], sem).wait()

        def process(b):
            sb = sbuf.at[b]
            db = dbuf.at[b]

            @plsc.parallel_loop(0, CH, step=16, unroll=4)
            def _(j):
                s = sb[pl.ds(j, 16)]
                t = db[pl.ds(j, 16)]
                ex = _leaky_exp(plsc.load_gather(es_v, [s]),
                                plsc.load_gather(ed_v, [t]))
                plsc.addupdate_scatter(ssum_v, [t], ex)
                for d in range(4):
                    hg = plsc.load_gather(h_v, [s + d * N])
                    plsc.addupdate_scatter(acc_v, [t + d * N], ex * hg)

        for p in range(4):
            head = 2 * p + wid // 16
            rb = p * 128 + wid * 4
            pltpu.sync_copy(esd_hbm.at[head, pl.ds(0, N)], es_v)
            pltpu.sync_copy(esd_hbm.at[H1 + head, pl.ds(0, N)], ed_v)
            for d in range(4):
                pltpu.sync_copy(h_hbm.at[rb + d, pl.ds(0, N)],
                                h_v.at[pl.ds(d * N, N)])
            fire(0, 0, sems[0])

            @plsc.parallel_loop(0, N, step=16, unroll=4)
            def _(j):
                ex = _leaky_exp(es_v[pl.ds(j, 16)], ed_v[pl.ds(j, 16)])
                ssum_v[pl.ds(j, 16)] = ex
                for d in range(4):
                    acc_v[pl.ds(j + d * N, 16)] = ex * h_v[pl.ds(j + d * N, 16)]

            @pl.loop(0, NCH1, step=2)
            def _(g):
                fire(g + 1, 1, sems[1])
                drain(g, 0, sems[0])
                process(0)

                @pl.when(g + 2 < NCH1)
                def _():
                    fire(g + 2, 0, sems[0])

                drain(g + 1, 1, sems[1])
                process(1)

            @plsc.parallel_loop(0, N, step=16, unroll=4)
            def _(j):
                inv = 1.0 / (ssum_v[pl.ds(j, 16)] + 1e-9)
                for d in range(4):
                    v = acc_v[pl.ds(j + d * N, 16)] * inv
                    acc_v[pl.ds(j + d * N, 16)] = jnp.where(
                        v > 0, v, jnp.exp(v) - 1.0)

            for d in range(4):
                pltpu.sync_copy(acc_v.at[pl.ds(d * N, N)],
                                out_hbm.at[rb + d, pl.ds(0, N)])

    return k(hT, esd, src, dst)


def _sc2(h2T, esd2, src, dst):
    @pl.kernel(
        out_type=[
            jax.ShapeDtypeStruct((2 * NLAB, NPAD), _F32),
            jax.ShapeDtypeStruct((2, NPAD), _F32),
        ],
        mesh=_SC_MESH,
        scratch_types=[
            pltpu.VMEM((N,), _F32),
            pltpu.VMEM((N,), _F32),
            pltpu.VMEM((N,), _F32),
            pltpu.VMEM((N,), _F32),
            pltpu.VMEM((N,), _F32),
            pltpu.VMEM((2, CH), jnp.int32),
            pltpu.VMEM((2, CH), jnp.int32),
            pltpu.SemaphoreType.DMA,
            pltpu.SemaphoreType.DMA,
        ],
        compiler_params=_SC_PARAMS,
    )
    def k(h_hbm, esd_hbm, src_hbm, dst_hbm, acc_hbm, ss_hbm,
          es_v, ed_v, h_v, acc_v, ssum_v, sbuf, dbuf, sem0, sem1):
        wid = lax.axis_index("s") * 2 + lax.axis_index("c")
        d = wid % NLAB
        half = wid // NLAB
        ebase = half * NCH2
        sems = (sem0, sem1)

        def fire(chunk, b, sem):
            off = (ebase + chunk) * CH
            pltpu.async_copy(src_hbm.at[pl.ds(off, CH)], sbuf.at[b], sem)
            pltpu.async_copy(dst_hbm.at[pl.ds(off, CH)], dbuf.at[b], sem)

        def drain(chunk, b, sem):
            off = (ebase + chunk) * CH
            pltpu.make_async_copy(src_hbm.at[pl.ds(off, CH)], sbuf.at[b], sem).wait()
            pltpu.make_async_copy(dst_hbm.at[pl.ds(off, CH)], dbuf.at[b], sem).wait()

        def process(b):
            sb = sbuf.at[b]
            db = dbuf.at[b]

            @plsc.parallel_loop(0, CH, step=16, unroll=4)
            def _(j):
                s = sb[pl.ds(j, 16)]
                t = db[pl.ds(j, 16)]
                ex = _leaky_exp(plsc.load_gather(es_v, [s]),
                                plsc.load_gather(ed_v, [t]))
                plsc.addupdate_scatter(ssum_v, [t], ex)
                hg = plsc.load_gather(h_v, [s])
                plsc.addupdate_scatter(acc_v, [t], ex * hg)

        pltpu.sync_copy(esd_hbm.at[0, pl.ds(0, N)], es_v)
        pltpu.sync_copy(esd_hbm.at[1, pl.ds(0, N)], ed_v)
        pltpu.sync_copy(h_hbm.at[d, pl.ds(0, N)], h_v)
        fire(0, 0, sems[0])

        flag = jnp.where(half == 0, 1.0, 0.0).astype(_F32)

        @plsc.parallel_loop(0, N, step=16, unroll=4)
        def _(j):
            ex = flag * _leaky_exp(es_v[pl.ds(j, 16)], ed_v[pl.ds(j, 16)])
            ssum_v[pl.ds(j, 16)] = ex
            acc_v[pl.ds(j, 16)] = ex * h_v[pl.ds(j, 16)]

        @pl.loop(0, NCH2, step=2)
        def _(g):
            fire(g + 1, 1, sems[1])
            drain(g, 0, sems[0])
            process(0)

            @pl.when(g + 2 < NCH2)
            def _():
                fire(g + 2, 0, sems[0])

            drain(g + 1, 1, sems[1])
            process(1)

        pltpu.sync_copy(acc_v, acc_hbm.at[wid, pl.ds(0, N)])

        @pl.when(d == 0)
        def _():
            pltpu.sync_copy(ssum_v, ss_hbm.at[half, pl.ds(0, N)])

    return k(h2T, esd2, src, dst)


def kernel(x, edge_index, W1, a1s, a1d, W2, a2s, a2d):
    xT = jnp.pad(x.T, ((0, 0), (0, NPAD - N)))
    src = edge_index[0]
    dst = edge_index[1]

    w1flat = jnp.transpose(W1, (0, 2, 1)).reshape(H1 * NHID, D)
    eye = jnp.eye(H1, dtype=_F32)
    bd_s = jnp.einsum("ij,jf->ijf", eye, a1s).reshape(H1, H1 * NHID)
    bd_d = jnp.einsum("ij,jf->ijf", eye, a1d).reshape(H1, H1 * NHID)
    bd = jnp.concatenate([bd_s, bd_d], axis=0)

    hT, esd = _tc1(w1flat, bd, xT)
    h1T = _sc1(hT, esd, src, dst)

    w2T = W2[0].T
    a2 = jnp.concatenate([a2s, a2d], axis=0)
    h2T, esd2 = _tc2(w2T, a2, h1T)
    acc2, ssum2 = _sc2(h2T, esd2, src, dst)

    out = _tc3(acc2, ssum2)
    return out[:N]
